# baseline (speedup 1.0000x reference)
import functools

import jax
import jax.numpy as jnp
from jax import lax
from jax.experimental import pallas as pl
from jax.experimental.pallas import tpu as pltpu
from jax.experimental.pallas import tpu_sc as plsc

DEPTH = 2000
ROWS = 1024
COLS = 20
FLAT = ROWS * COLS * DEPTH
SLAB = (DEPTH // 8) * 8 * ROWS
ZCH = 80000
NZD = 16
RGROUPS = ROWS // 16
NIB = 16

_info = plsc.get_sparse_core_info()
_NC, _NS = _info.num_cores, _info.num_subcores
_NW = _NC * _NS
_TW = FLAT // _NW

_mesh = plsc.VectorSubcoreMesh(core_axis_name="c", subcore_axis_name="s")


@functools.partial(
    pl.kernel,
    mesh=_mesh,
    out_type=jax.ShapeDtypeStruct((FLAT,), jnp.float32),
    scratch_types=[
        pltpu.VMEM((ZCH,), jnp.float32),
        [pltpu.VMEM((ROWS,), jnp.int32) for _ in range(2)],
        [pltpu.VMEM((128,), jnp.int32) for _ in range(NIB)],
        [pltpu.VMEM((128,), jnp.float32) for _ in range(NIB)],
        pltpu.SemaphoreType.DMA,
        pltpu.SemaphoreType.DMA,
    ],
    compiler_params=pltpu.CompilerParams(needs_layout_passes=False,
                                         use_tc_tiling_on_sc=True),
)
def _onehot_sc(idx_hbm, zeros_hbm, out_hbm, zbuf, idx_vs, pos_vs, val_vs,
               zsem, ssem):
    wid = lax.axis_index("s") * _NC + lax.axis_index("c")
    lo = wid * _TW
    hi = lo + _TW
    pltpu.sync_copy(zeros_hbm, zbuf)

    zcopies = [
        pltpu.async_copy(zbuf, out_hbm.at[pl.ds(lo + i * ZCH, ZCH)], zsem)
        for i in range(NZD)
    ]

    lane = lax.iota(jnp.int32, 16)
    j0 = lo // SLAB
    for jj in range(2):
        jv = j0 + jj
        valid = jv * SLAB < hi
        jl = jnp.where(valid, jv, j0)
        idx_v = idx_vs[jj]
        pltpu.sync_copy(idx_hbm.at[pl.ds(jl * ROWS, ROWS)], idx_v)
        for g in range(RGROUPS):
            r = g * 16 + lane
            d = idx_v[pl.ds(g * 16, 16)]
            w = jl * SLAB + (d // 8) * 8192 + (r // 128) * 1024 \
                + (d % 8) * 128 + (r % 128)
            match = (w >= lo) & (w < hi) & valid
            d_eff = jnp.where(match, d, d ^ 1)
            pos = jl * SLAB + (d_eff // 8) * 8192 + (r // 128) * 1024 \
                + (d_eff % 8) * 128 + (r % 128)
            val = jnp.where(match, 1.0, 0.0).astype(jnp.float32)
            gg = jj * RGROUPS + g
            pos_vs[gg // 8][pl.ds((gg % 8) * 16, 16)] = pos
            val_vs[gg // 8][pl.ds((gg % 8) * 16, 16)] = val

    for c in zcopies:
        c.wait()
    scopies = [
        pltpu.async_copy(val_vs[b], out_hbm.at[pos_vs[b]], ssem)
        for b in range(NIB)
    ]
    for c in scopies:
        c.wait()


def kernel(inputs):
    idx_t = inputs.astype(jnp.int32).T.reshape(-1)
    zeros = jnp.zeros((ZCH,), jnp.float32)
    flat = _onehot_sc(idx_t, zeros)
    z5 = flat.reshape(COLS, DEPTH // 8, ROWS // 128, 8, 128)
    return z5.transpose(2, 4, 0, 1, 3).reshape(ROWS, COLS, DEPTH)


# speedup vs baseline: 1.1425x; 1.1425x over previous
import functools

import jax
import jax.numpy as jnp
from jax import lax
from jax.experimental import pallas as pl
from jax.experimental.pallas import tpu as pltpu
from jax.experimental.pallas import tpu_sc as plsc

DEPTH = 2000
ROWS = 1024
COLS = 20
DC = 40
NWIN = DEPTH // DC
NUNITS = COLS * NWIN
RGROUPS = ROWS // 16

_info = plsc.get_sparse_core_info()
_NC, _NS = _info.num_cores, _info.num_subcores
_NW = _NC * _NS
_UPT = -(-NUNITS // _NW)

_mesh = plsc.VectorSubcoreMesh(core_axis_name="c", subcore_axis_name="s")


@functools.partial(
    pl.kernel,
    mesh=_mesh,
    out_type=jax.ShapeDtypeStruct((COLS, DEPTH, ROWS), jnp.float32),
    scratch_types=[
        pltpu.VMEM((ROWS,), jnp.int32),
        [pltpu.VMEM((1, DC, ROWS), jnp.float32) for _ in range(2)],
        [pltpu.VMEM((ROWS,), jnp.int32) for _ in range(2)],
        [pltpu.SemaphoreType.DMA for _ in range(2)],
    ],
    compiler_params=pltpu.CompilerParams(needs_layout_passes=False,
                                         use_tc_tiling_on_sc=True),
)
def _onehot_sc(idx_hbm, zeros_hbm, out_hbm, idx_v, bufs, recs, sems):
    wid = lax.axis_index("s") * _NC + lax.axis_index("c")

    lane = lax.iota(jnp.int32, 16)
    z16 = jnp.zeros((16,), jnp.int32)
    ones_f = jnp.ones((16,), jnp.float32)
    zeros_f = jnp.zeros((16,), jnp.float32)

    pltpu.sync_copy(zeros_hbm, bufs[0])
    pltpu.sync_copy(zeros_hbm, bufs[1])

    def drain(b):
        pltpu.make_async_copy(
            bufs[b], out_hbm.at[pl.ds(0, 1), pl.ds(0, DC)], sems[b]).wait()

    for k in range(_UPT):
        b = k % 2
        u = wid + k * _NW

        @pl.when(u < NUNITS)
        def _(b=b, k=k, u=u):
            j = u // NWIN
            d0 = (u % NWIN) * DC
            pltpu.sync_copy(idx_hbm.at[pl.ds(j * ROWS, ROWS)], idx_v)

            if k >= 2:
                drain(b)

                def clr_body(g, c):
                    dd = recs[b][pl.ds(g * 16, 16)]
                    match = dd >= 0
                    d_id = jnp.clip(dd, 0, DC - 1)
                    plsc.store_scatter(bufs[b], [z16, d_id, g * 16 + lane],
                                       zeros_f, mask=match)
                    return c

                lax.fori_loop(0, RGROUPS, clr_body, 0)

            def set_body(g, c):
                d = idx_v[pl.ds(g * 16, 16)]
                dd = d - d0
                match = (dd >= 0) & (dd < DC)
                d_id = jnp.clip(dd, 0, DC - 1)
                plsc.store_scatter(bufs[b], [z16, d_id, g * 16 + lane],
                                   ones_f, mask=match)
                recs[b][pl.ds(g * 16, 16)] = jnp.where(match, dd, -1)
                return c

            lax.fori_loop(0, RGROUPS, set_body, 0)
            pltpu.async_copy(
                bufs[b], out_hbm.at[pl.ds(j, 1), pl.ds(d0, DC)], sems[b])

    for kk in range(_UPT - 3, _UPT):
        u = wid + kk * _NW

        @pl.when((u < NUNITS) & (u + 2 * _NW >= NUNITS))
        def _(kk=kk):
            drain(kk % 2)


def kernel(inputs):
    idx_t = inputs.astype(jnp.int32).T.reshape(-1)
    zeros = jnp.zeros((1, DC, ROWS), jnp.float32)
    out = _onehot_sc(idx_t, zeros)
    return out.transpose(2, 0, 1)


# speedup vs baseline: 1.6101x; 1.4092x over previous
import functools

import jax
import jax.numpy as jnp
from jax import lax
from jax.experimental import pallas as pl
from jax.experimental.pallas import tpu as pltpu
from jax.experimental.pallas import tpu_sc as plsc

DEPTH = 2000
ROWS = 1024
COLS = 20
DC = 40
NWIN = DEPTH // DC
NUNITS = COLS * NWIN
RGROUPS = ROWS // 16

_info = plsc.get_sparse_core_info()
_NC, _NS = _info.num_cores, _info.num_subcores
_NW = _NC * _NS
_UPT = -(-NUNITS // _NW)

_mesh = plsc.VectorSubcoreMesh(core_axis_name="c", subcore_axis_name="s")


@functools.partial(
    pl.kernel,
    mesh=_mesh,
    out_type=jax.ShapeDtypeStruct((COLS, DEPTH, ROWS), jnp.float32),
    scratch_types=[
        pltpu.VMEM((ROWS * COLS,), jnp.int32),
        [pltpu.VMEM((1, DC, ROWS), jnp.float32) for _ in range(2)],
        [pltpu.VMEM((ROWS,), jnp.int32) for _ in range(2)],
        [pltpu.SemaphoreType.DMA for _ in range(2)],
    ],
    compiler_params=pltpu.CompilerParams(needs_layout_passes=False,
                                         use_tc_tiling_on_sc=True),
)
def _onehot_sc(idx_hbm, zeros_hbm, out_hbm, idx_v, bufs, recs, sems):
    wid = lax.axis_index("s") * _NC + lax.axis_index("c")

    lane = lax.iota(jnp.int32, 16)
    z16 = jnp.zeros((16,), jnp.int32)
    ones_f = jnp.ones((16,), jnp.float32)
    zeros_f = jnp.zeros((16,), jnp.float32)

    pltpu.sync_copy(zeros_hbm, bufs[0])
    pltpu.sync_copy(zeros_hbm, bufs[1])
    pltpu.sync_copy(idx_hbm, idx_v)

    def drain(b):
        pltpu.make_async_copy(
            bufs[b], out_hbm.at[pl.ds(0, 1), pl.ds(0, DC)], sems[b]).wait()

    for k in range(_UPT):
        b = k % 2
        u = wid + k * _NW

        @pl.when(u < NUNITS)
        def _(b=b, k=k, u=u):
            j = u // NWIN
            d0 = (u % NWIN) * DC

            if k >= 2:
                drain(b)

                def clr_body(g, c):
                    dd = recs[b][pl.ds(g * 16, 16)]
                    match = dd >= 0
                    d_id = jnp.clip(dd, 0, DC - 1)
                    plsc.store_scatter(bufs[b], [z16, d_id, g * 16 + lane],
                                       zeros_f, mask=match)
                    return c

                lax.fori_loop(0, RGROUPS, clr_body, 0)

            def set_body(g, c):
                d = idx_v[pl.ds(j * ROWS + g * 16, 16)]
                dd = d - d0
                match = (dd >= 0) & (dd < DC)
                d_id = jnp.clip(dd, 0, DC - 1)
                plsc.store_scatter(bufs[b], [z16, d_id, g * 16 + lane],
                                   ones_f, mask=match)
                recs[b][pl.ds(g * 16, 16)] = jnp.where(match, dd, -1)
                return c

            lax.fori_loop(0, RGROUPS, set_body, 0)
            pltpu.async_copy(
                bufs[b], out_hbm.at[pl.ds(j, 1), pl.ds(d0, DC)], sems[b])

    for kk in range(_UPT - 3, _UPT):
        u = wid + kk * _NW

        @pl.when((u < NUNITS) & (u + 2 * _NW >= NUNITS))
        def _(kk=kk):
            drain(kk % 2)


def kernel(inputs):
    idx_t = inputs.astype(jnp.int32).T.reshape(-1)
    zeros = jnp.zeros((1, DC, ROWS), jnp.float32)
    out = _onehot_sc(idx_t, zeros)
    return out.transpose(2, 0, 1)
